# baseline (speedup 1.0000x reference)
import jax
import jax.numpy as jnp
from jax import lax
from jax.experimental import pallas as pl
from jax.experimental.pallas import tpu as pltpu
from jax.experimental.pallas import tpu_sc as plsc

_CHUNK = 128
_NB = 4
_F = 32


def _relu(v):
    return jnp.maximum(v, 0.0)


def _seg_sum_sc(y, src3, dst3, zeros, n_pad, n_chunks, nc, ns, f, nb):
    rpt = n_pad // ns

    mesh = plsc.VectorSubcoreMesh(core_axis_name="c", subcore_axis_name="s")

    n_groups = n_chunks // nb

    def body(y_hbm, src_hbm, dst_hbm, z_hbm, out_hbm,
             src_v, dst_v, rows0, rows1, acc_sh,
             gsem0, gsem1, ssem0, ssem1):
        c = lax.axis_index("c")
        s = lax.axis_index("s")
        wid = c * ns + s
        pltpu.sync_copy(z_hbm.at[pl.ds(s * rpt, rpt)],
                        acc_sh.at[pl.ds(s * rpt, rpt)])
        pltpu.sync_copy(src_hbm.at[wid], src_v)
        pltpu.sync_copy(dst_hbm.at[wid], dst_v)
        plsc.subcore_barrier()

        def gath(j, rows, gsem, b):
            pltpu.async_copy(y_hbm.at[src_v.at[j]], rows.at[b], gsem)

        def scat(j, rows, ssem, b):
            pltpu.async_copy(rows.at[b], acc_sh.at[dst_v.at[j]], ssem, add=True)

        def gwait(rows, gsem, b):
            pltpu.make_async_copy(y_hbm.at[src_v.at[0]], rows.at[b], gsem).wait()

        def swait(rows, ssem, b):
            pltpu.make_async_copy(rows.at[b], acc_sh.at[dst_v.at[0]], ssem).wait()

        for b in range(nb):
            gath(b, rows0, gsem0, b)

        def pair(t, carry):
            g0 = 2 * t
            @pl.when(t > 0)
            def _():
                for b in range(nb):
                    swait(rows1, ssem1, b)
            for b in range(nb):
                gath((g0 + 1) * nb + b, rows1, gsem1, b)
            for b in range(nb):
                gwait(rows0, gsem0, b)
                scat(g0 * nb + b, rows0, ssem0, b)
            for b in range(nb):
                swait(rows0, ssem0, b)
            @pl.when(t < n_groups // 2 - 1)
            def _():
                for b in range(nb):
                    gath((g0 + 2) * nb + b, rows0, gsem0, b)
            for b in range(nb):
                gwait(rows1, gsem1, b)
                scat((g0 + 1) * nb + b, rows1, ssem1, b)
            return carry

        lax.fori_loop(0, n_groups // 2, pair, 0)
        for b in range(nb):
            swait(rows1, ssem1, b)
        plsc.subcore_barrier()
        pltpu.sync_copy(acc_sh.at[pl.ds(s * rpt, rpt)],
                        out_hbm.at[c, pl.ds(s * rpt, rpt)])

    f = pl.kernel(
        body,
        out_type=jax.ShapeDtypeStruct((nc, n_pad, f), jnp.float32),
        mesh=mesh,
        scratch_types=[
            pltpu.VMEM((n_chunks, _CHUNK), jnp.int32),
            pltpu.VMEM((n_chunks, _CHUNK), jnp.int32),
            pltpu.VMEM((nb, _CHUNK, f), jnp.float32),
            pltpu.VMEM((nb, _CHUNK, f), jnp.float32),
            pltpu.VMEM_SHARED((n_pad, f), jnp.float32),
            pltpu.SemaphoreType.DMA,
            pltpu.SemaphoreType.DMA,
            pltpu.SemaphoreType.DMA,
            pltpu.SemaphoreType.DMA,
        ],
        compiler_params=pltpu.CompilerParams(use_tc_tiling_on_sc=False),
    )
    return f(y, src3, dst3, zeros)


def _tc_pre(x, w1, b1, w2, b2, w3, b3, rblk):
    n_pad, din = x.shape

    def body(x_ref, w1_ref, b1_ref, w2_ref, b2_ref, w3_ref, b3_ref, s_ref):
        xv = x_ref[...]
        h1 = jnp.dot(xv, w1_ref[...], preferred_element_type=jnp.float32) + b1_ref[...]
        h2 = jnp.dot(xv, w2_ref[...], preferred_element_type=jnp.float32) + b2_ref[...]
        h3 = jnp.dot(xv, w3_ref[...], preferred_element_type=jnp.float32) + b3_ref[...]
        s_ref[...] = _relu(h1) + _relu(h2 * h3)

    wspec = pl.BlockSpec((din, _F), lambda i: (0, 0))
    bspec = pl.BlockSpec((1, _F), lambda i: (0, 0))
    return pl.pallas_call(
        body,
        grid=(n_pad // rblk,),
        in_specs=[pl.BlockSpec((rblk, din), lambda i: (i, 0)),
                  wspec, bspec, wspec, bspec, wspec, bspec],
        out_specs=pl.BlockSpec((rblk, _F), lambda i: (i, 0)),
        out_shape=jax.ShapeDtypeStruct((n_pad, _F), jnp.float32),
    )(x, w1, b1, w2, b2, w3, b3)


def _tc_mid(s_prev, ps, wc_prev, cb, w1, b1, w2, b2, w3, b3, rblk):
    n_pad = s_prev.shape[0]
    nps = len(ps)

    def body(*refs):
        s_ref = refs[0]
        p_refs = refs[1:1 + nps]
        (wcp_ref, cb_ref, w1_ref, b1_ref, w2_ref, b2_ref, w3_ref, b3_ref,
         x_ref, s2_ref) = refs[1 + nps:]
        agg = jnp.concatenate([p[0] + p[1] for p in p_refs], axis=1)
        agg = jnp.dot(agg, wcp_ref[...], preferred_element_type=jnp.float32)
        xk = s_ref[...] + _relu(agg + cb_ref[...])
        h1 = jnp.dot(xk, w1_ref[...], preferred_element_type=jnp.float32) + b1_ref[...]
        h2 = jnp.dot(xk, w2_ref[...], preferred_element_type=jnp.float32) + b2_ref[...]
        h3 = jnp.dot(xk, w3_ref[...], preferred_element_type=jnp.float32) + b3_ref[...]
        x_ref[...] = xk
        s2_ref[...] = _relu(h1) + _relu(h2 * h3)

    fin = sum(p.shape[-1] for p in ps)
    wspec = pl.BlockSpec((_F, _F), lambda i: (0, 0))
    bspec = pl.BlockSpec((1, _F), lambda i: (0, 0))

    def pspec(f):
        return pl.BlockSpec((2, rblk, f), lambda i: (0, i, 0))

    return pl.pallas_call(
        body,
        grid=(n_pad // rblk,),
        in_specs=[pl.BlockSpec((rblk, _F), lambda i: (i, 0))]
                 + [pspec(p.shape[-1]) for p in ps]
                 + [pl.BlockSpec((fin, _F), lambda i: (0, 0)),
                    bspec, wspec, bspec, wspec, bspec, wspec, bspec],
        out_specs=[pl.BlockSpec((rblk, _F), lambda i: (i, 0)),
                   pl.BlockSpec((rblk, _F), lambda i: (i, 0))],
        out_shape=[jax.ShapeDtypeStruct((n_pad, _F), jnp.float32),
                   jax.ShapeDtypeStruct((n_pad, _F), jnp.float32)],
    )(s_prev, *ps, wc_prev, cb, w1, b1, w2, b2, w3, b3)


def _tc_fin(s_prev, p, wc_prev, cb, w2, b2, rblk):
    n_pad = s_prev.shape[0]

    def body(s_ref, p_ref, wcp_ref, cb_ref, w_ref, b_ref, o_ref):
        agg = jnp.dot(p_ref[0] + p_ref[1], wcp_ref[...],
                      preferred_element_type=jnp.float32)
        xk = s_ref[...] + _relu(agg + cb_ref[...])
        o_ref[...] = jnp.dot(xk, w_ref[...],
                             preferred_element_type=jnp.float32) + b_ref[...]

    return pl.pallas_call(
        body,
        grid=(n_pad // rblk,),
        in_specs=[pl.BlockSpec((rblk, _F), lambda i: (i, 0)),
                  pl.BlockSpec((2, rblk, _F), lambda i: (0, i, 0)),
                  pl.BlockSpec((_F, _F), lambda i: (0, 0)),
                  pl.BlockSpec((1, _F), lambda i: (0, 0)),
                  pl.BlockSpec((_F, 1), lambda i: (0, 0)),
                  pl.BlockSpec((1, 1), lambda i: (0, 0))],
        out_specs=pl.BlockSpec((rblk, 1), lambda i: (i, 0)),
        out_shape=jax.ShapeDtypeStruct((n_pad, 1), jnp.float32),
    )(s_prev, p, wc_prev, cb, w2, b2)


def kernel(x, edge_index,
           conv11_w, conv11_b, conv21_w, conv21_b, conv31_w, conv31_b,
           fc11_w, fc11_b, fc12_w, fc12_b, fc13_w, fc13_b,
           fc21_w, fc21_b, fc22_w, fc22_b, fc23_w, fc23_b,
           fc31_w, fc31_b, fc32_w, fc32_b, fc33_w, fc33_b,
           fc2_w, fc2_b):
    n, din = x.shape
    e = edge_index.shape[1]
    info = plsc.get_sparse_core_info()
    nc, ns = info.num_cores, info.num_subcores
    nw = nc * ns

    n_chunks = -(-e // (nw * _CHUNK * 2 * _NB)) * (2 * _NB)
    e_pad = nw * _CHUNK * n_chunks
    n_pad = -(-n // (8 * ns)) * (8 * ns)
    if e_pad > e and n_pad == n:
        n_pad += 8 * ns
    rblk = n_pad // 4

    src = edge_index[0]
    dst = edge_index[1]
    pad = e_pad - e
    if pad:
        src = jnp.concatenate([src, jnp.zeros((pad,), jnp.int32)])
        dst = jnp.concatenate([dst, jnp.full((pad,), n, jnp.int32)])
    src3 = src.reshape(nw, n_chunks, _CHUNK)
    dst3 = dst.reshape(nw, n_chunks, _CHUNK)

    xp = jnp.pad(x, ((0, n_pad - n), (0, 0)))
    zeros_f = jnp.zeros((n_pad, _F), jnp.float32)

    def row(b):
        return b.reshape(1, -1)

    s1 = _tc_pre(xp, fc11_w, row(fc11_b), fc12_w, row(fc12_b),
                 fc13_w, row(fc13_b), rblk)
    dh = din // 2
    zeros_h = jnp.zeros((n_pad, dh), jnp.float32)
    p1a = _seg_sum_sc(xp[:, :dh], src3, dst3, zeros_h, n_pad, n_chunks,
                      nc, ns, dh, _NB)
    p1b = _seg_sum_sc(xp[:, dh:], src3, dst3, zeros_h, n_pad, n_chunks,
                      nc, ns, dh, _NB)
    x1, s2 = _tc_mid(s1, [p1a, p1b], conv11_w, row(conv11_b),
                     fc21_w, row(fc21_b), fc22_w, row(fc22_b),
                     fc23_w, row(fc23_b), rblk)
    p2 = _seg_sum_sc(x1, src3, dst3, zeros_f, n_pad, n_chunks, nc, ns, _F, _NB)
    x2, s3 = _tc_mid(s2, [p2], conv21_w, row(conv21_b),
                     fc31_w, row(fc31_b), fc32_w, row(fc32_b),
                     fc33_w, row(fc33_b), rblk)
    p3 = _seg_sum_sc(x2, src3, dst3, zeros_f, n_pad, n_chunks, nc, ns, _F, _NB)
    out = _tc_fin(s3, p3, conv31_w, row(conv31_b), fc2_w,
                  fc2_b.reshape(1, 1), rblk)
    return out[:n]


# speedup vs baseline: 1.0709x; 1.0709x over previous
import jax
import jax.numpy as jnp
from jax import lax
from jax.experimental import pallas as pl
from jax.experimental.pallas import tpu as pltpu
from jax.experimental.pallas import tpu_sc as plsc

_CHUNK = 128
_NB = 4
_F = 32


def _relu(v):
    return jnp.maximum(v, 0.0)


def _seg_sum_sc(y, src3, dst3, n_pad, n_chunks, nc, ns, f, nb):
    rpt = n_pad // ns

    mesh = plsc.VectorSubcoreMesh(core_axis_name="c", subcore_axis_name="s")

    n_groups = n_chunks // nb

    def body(y_hbm, src_hbm, dst_hbm, out_hbm,
             src_v, dst_v, rows0, rows1, zbuf, acc_sh,
             gsem0, gsem1, ssem0, ssem1):
        c = lax.axis_index("c")
        s = lax.axis_index("s")
        wid = c * ns + s

        zb = 128

        def zrow(r, carry):
            for k in range(f // 16):
                zbuf[r, pl.ds(16 * k, 16)] = jnp.zeros((16,), jnp.float32)
            return carry

        lax.fori_loop(0, zb, zrow, 0)
        off = 0
        while off < rpt:
            step = min(zb, rpt - off)
            pltpu.sync_copy(zbuf.at[pl.ds(0, step)],
                            acc_sh.at[pl.ds(s * rpt + off, step)])
            off += step

        pltpu.sync_copy(src_hbm.at[wid], src_v)
        pltpu.sync_copy(dst_hbm.at[wid], dst_v)
        plsc.subcore_barrier()

        def gath(j, rows, gsem, b):
            pltpu.async_copy(y_hbm.at[src_v.at[j]], rows.at[b], gsem)

        def scat(j, rows, ssem, b):
            pltpu.async_copy(rows.at[b], acc_sh.at[dst_v.at[j]], ssem, add=True)

        def gwait(rows, gsem, b):
            pltpu.make_async_copy(y_hbm.at[src_v.at[0]], rows.at[b], gsem).wait()

        def swait(rows, ssem, b):
            pltpu.make_async_copy(rows.at[b], acc_sh.at[dst_v.at[0]], ssem).wait()

        for b in range(nb):
            gath(b, rows0, gsem0, b)

        def pair(t, carry):
            g0 = 2 * t
            @pl.when(t > 0)
            def _():
                for b in range(nb):
                    swait(rows1, ssem1, b)
            for b in range(nb):
                gath((g0 + 1) * nb + b, rows1, gsem1, b)
            for b in range(nb):
                gwait(rows0, gsem0, b)
                scat(g0 * nb + b, rows0, ssem0, b)
            for b in range(nb):
                swait(rows0, ssem0, b)
            @pl.when(t < n_groups // 2 - 1)
            def _():
                for b in range(nb):
                    gath((g0 + 2) * nb + b, rows0, gsem0, b)
            for b in range(nb):
                gwait(rows1, gsem1, b)
                scat((g0 + 1) * nb + b, rows1, ssem1, b)
            return carry

        lax.fori_loop(0, n_groups // 2, pair, 0)
        for b in range(nb):
            swait(rows1, ssem1, b)
        plsc.subcore_barrier()
        pltpu.sync_copy(acc_sh.at[pl.ds(s * rpt, rpt)],
                        out_hbm.at[c, pl.ds(s * rpt, rpt)])

    f_k = pl.kernel(
        body,
        out_type=jax.ShapeDtypeStruct((nc, n_pad, f), jnp.float32),
        mesh=mesh,
        scratch_types=[
            pltpu.VMEM((n_chunks, _CHUNK), jnp.int32),
            pltpu.VMEM((n_chunks, _CHUNK), jnp.int32),
            pltpu.VMEM((nb, _CHUNK, f), jnp.float32),
            pltpu.VMEM((nb, _CHUNK, f), jnp.float32),
            pltpu.VMEM((128, f), jnp.float32),
            pltpu.VMEM_SHARED((n_pad, f), jnp.float32),
            pltpu.SemaphoreType.DMA,
            pltpu.SemaphoreType.DMA,
            pltpu.SemaphoreType.DMA,
            pltpu.SemaphoreType.DMA,
        ],
        compiler_params=pltpu.CompilerParams(use_tc_tiling_on_sc=False),
    )
    return f_k(y, src3, dst3)


def _tc_pre(x, w1, b1, w2, b2, w3, b3, rblk):
    n_pad, din = x.shape

    def body(x_ref, w1_ref, b1_ref, w2_ref, b2_ref, w3_ref, b3_ref, s_ref):
        xv = x_ref[...]
        h1 = jnp.dot(xv, w1_ref[...], preferred_element_type=jnp.float32) + b1_ref[...]
        h2 = jnp.dot(xv, w2_ref[...], preferred_element_type=jnp.float32) + b2_ref[...]
        h3 = jnp.dot(xv, w3_ref[...], preferred_element_type=jnp.float32) + b3_ref[...]
        s_ref[...] = _relu(h1) + _relu(h2 * h3)

    wspec = pl.BlockSpec((din, _F), lambda i: (0, 0))
    bspec = pl.BlockSpec((1, _F), lambda i: (0, 0))
    return pl.pallas_call(
        body,
        grid=(n_pad // rblk,),
        in_specs=[pl.BlockSpec((rblk, din), lambda i: (i, 0)),
                  wspec, bspec, wspec, bspec, wspec, bspec],
        out_specs=pl.BlockSpec((rblk, _F), lambda i: (i, 0)),
        out_shape=jax.ShapeDtypeStruct((n_pad, _F), jnp.float32),
    )(x, w1, b1, w2, b2, w3, b3)


def _tc_mid(s_prev, ps, wc_prev, cb, w1, b1, w2, b2, w3, b3, rblk):
    n_pad = s_prev.shape[0]
    nps = len(ps)

    def body(*refs):
        s_ref = refs[0]
        p_refs = refs[1:1 + nps]
        (wcp_ref, cb_ref, w1_ref, b1_ref, w2_ref, b2_ref, w3_ref, b3_ref,
         x_ref, s2_ref) = refs[1 + nps:]
        agg = jnp.concatenate([p[0] + p[1] for p in p_refs], axis=1)
        agg = jnp.dot(agg, wcp_ref[...], preferred_element_type=jnp.float32)
        xk = s_ref[...] + _relu(agg + cb_ref[...])
        h1 = jnp.dot(xk, w1_ref[...], preferred_element_type=jnp.float32) + b1_ref[...]
        h2 = jnp.dot(xk, w2_ref[...], preferred_element_type=jnp.float32) + b2_ref[...]
        h3 = jnp.dot(xk, w3_ref[...], preferred_element_type=jnp.float32) + b3_ref[...]
        x_ref[...] = xk
        s2_ref[...] = _relu(h1) + _relu(h2 * h3)

    fin = sum(p.shape[-1] for p in ps)
    wspec = pl.BlockSpec((_F, _F), lambda i: (0, 0))
    bspec = pl.BlockSpec((1, _F), lambda i: (0, 0))

    def pspec(f):
        return pl.BlockSpec((2, rblk, f), lambda i: (0, i, 0))

    return pl.pallas_call(
        body,
        grid=(n_pad // rblk,),
        in_specs=[pl.BlockSpec((rblk, _F), lambda i: (i, 0))]
                 + [pspec(p.shape[-1]) for p in ps]
                 + [pl.BlockSpec((fin, _F), lambda i: (0, 0)),
                    bspec, wspec, bspec, wspec, bspec, wspec, bspec],
        out_specs=[pl.BlockSpec((rblk, _F), lambda i: (i, 0)),
                   pl.BlockSpec((rblk, _F), lambda i: (i, 0))],
        out_shape=[jax.ShapeDtypeStruct((n_pad, _F), jnp.float32),
                   jax.ShapeDtypeStruct((n_pad, _F), jnp.float32)],
    )(s_prev, *ps, wc_prev, cb, w1, b1, w2, b2, w3, b3)


def _tc_fin(s_prev, p, wc_prev, cb, w2, b2, rblk):
    n_pad = s_prev.shape[0]

    def body(s_ref, p_ref, wcp_ref, cb_ref, w_ref, b_ref, o_ref):
        agg = jnp.dot(p_ref[0] + p_ref[1], wcp_ref[...],
                      preferred_element_type=jnp.float32)
        xk = s_ref[...] + _relu(agg + cb_ref[...])
        o_ref[...] = jnp.dot(xk, w_ref[...],
                             preferred_element_type=jnp.float32) + b_ref[...]

    return pl.pallas_call(
        body,
        grid=(n_pad // rblk,),
        in_specs=[pl.BlockSpec((rblk, _F), lambda i: (i, 0)),
                  pl.BlockSpec((2, rblk, _F), lambda i: (0, i, 0)),
                  pl.BlockSpec((_F, _F), lambda i: (0, 0)),
                  pl.BlockSpec((1, _F), lambda i: (0, 0)),
                  pl.BlockSpec((_F, 1), lambda i: (0, 0)),
                  pl.BlockSpec((1, 1), lambda i: (0, 0))],
        out_specs=pl.BlockSpec((rblk, 1), lambda i: (i, 0)),
        out_shape=jax.ShapeDtypeStruct((n_pad, 1), jnp.float32),
    )(s_prev, p, wc_prev, cb, w2, b2)


def kernel(x, edge_index,
           conv11_w, conv11_b, conv21_w, conv21_b, conv31_w, conv31_b,
           fc11_w, fc11_b, fc12_w, fc12_b, fc13_w, fc13_b,
           fc21_w, fc21_b, fc22_w, fc22_b, fc23_w, fc23_b,
           fc31_w, fc31_b, fc32_w, fc32_b, fc33_w, fc33_b,
           fc2_w, fc2_b):
    n, din = x.shape
    e = edge_index.shape[1]
    info = plsc.get_sparse_core_info()
    nc, ns = info.num_cores, info.num_subcores
    nw = nc * ns

    n_chunks = -(-e // (nw * _CHUNK * 2 * _NB)) * (2 * _NB)
    e_pad = nw * _CHUNK * n_chunks
    n_pad = -(-n // (8 * ns)) * (8 * ns)
    if e_pad > e and n_pad == n:
        n_pad += 8 * ns
    rblk = n_pad // 4

    src = edge_index[0]
    dst = edge_index[1]
    pad = e_pad - e
    if pad:
        src = jnp.concatenate([src, jnp.zeros((pad,), jnp.int32)])
        dst = jnp.concatenate([dst, jnp.full((pad,), n, jnp.int32)])
    src3 = src.reshape(nw, n_chunks, _CHUNK)
    dst3 = dst.reshape(nw, n_chunks, _CHUNK)

    xp = jnp.pad(x, ((0, n_pad - n), (0, 0)))

    def row(b):
        return b.reshape(1, -1)

    s1 = _tc_pre(xp, fc11_w, row(fc11_b), fc12_w, row(fc12_b),
                 fc13_w, row(fc13_b), rblk)
    p1s = [_seg_sum_sc(xp[:, q * _F:(q + 1) * _F], src3, dst3, n_pad,
                       n_chunks, nc, ns, _F, _NB) for q in range(din // _F)]
    x1, s2 = _tc_mid(s1, p1s, conv11_w, row(conv11_b),
                     fc21_w, row(fc21_b), fc22_w, row(fc22_b),
                     fc23_w, row(fc23_b), rblk)
    p2 = _seg_sum_sc(x1, src3, dst3, n_pad, n_chunks, nc, ns, _F, _NB)
    x2, s3 = _tc_mid(s2, [p2], conv21_w, row(conv21_b),
                     fc31_w, row(fc31_b), fc32_w, row(fc32_b),
                     fc33_w, row(fc33_b), rblk)
    p3 = _seg_sum_sc(x2, src3, dst3, n_pad, n_chunks, nc, ns, _F, _NB)
    out = _tc_fin(s3, p3, conv31_w, row(conv31_b), fc2_w,
                  fc2_b.reshape(1, 1), rblk)
    return out[:n]


# speedup vs baseline: 2.4116x; 2.2519x over previous
import jax
import jax.numpy as jnp
from jax import lax
from jax.experimental import pallas as pl
from jax.experimental.pallas import tpu as pltpu
from jax.experimental.pallas import tpu_sc as plsc

_CHUNK = 128
_NB = 4
_F = 32


def _relu(v):
    return jnp.maximum(v, 0.0)


def _seg_sum_sc(y, src3, dst3, n_pad, n_chunks, nc, ns, f, nb):
    rpt = n_pad // ns

    mesh = plsc.VectorSubcoreMesh(core_axis_name="c", subcore_axis_name="s")

    n_groups = n_chunks // nb

    def body(y_hbm, src_hbm, dst_hbm, out_hbm,
             src_v, dst_v, rows0, rows1, zbuf, acc_sh, tab_sh,
             gsem0, gsem1, ssem0, ssem1):
        c = lax.axis_index("c")
        s = lax.axis_index("s")
        wid = c * ns + s

        zb = 128

        def zrow(r, carry):
            for k in range(f // 16):
                zbuf[r, pl.ds(16 * k, 16)] = jnp.zeros((16,), jnp.float32)
            return carry

        lax.fori_loop(0, zb, zrow, 0)
        off = 0
        while off < rpt:
            step = min(zb, rpt - off)
            pltpu.sync_copy(zbuf.at[pl.ds(0, step)],
                            acc_sh.at[pl.ds(s * rpt + off, step)])
            off += step

        pltpu.sync_copy(y_hbm.at[pl.ds(s * rpt, rpt)],
                        tab_sh.at[pl.ds(s * rpt, rpt)])
        pltpu.sync_copy(src_hbm.at[wid], src_v)
        pltpu.sync_copy(dst_hbm.at[wid], dst_v)
        plsc.subcore_barrier()

        def gath(j, rows, gsem, b):
            pltpu.async_copy(tab_sh.at[src_v.at[j]], rows.at[b], gsem)

        def scat(j, rows, ssem, b):
            pltpu.async_copy(rows.at[b], acc_sh.at[dst_v.at[j]], ssem, add=True)

        def gwait(rows, gsem, b):
            pltpu.make_async_copy(tab_sh.at[src_v.at[0]], rows.at[b], gsem).wait()

        def swait(rows, ssem, b):
            pltpu.make_async_copy(rows.at[b], acc_sh.at[dst_v.at[0]], ssem).wait()

        for b in range(nb):
            gath(b, rows0, gsem0, b)

        def pair(t, carry):
            g0 = 2 * t
            @pl.when(t > 0)
            def _():
                for b in range(nb):
                    swait(rows1, ssem1, b)
            for b in range(nb):
                gath((g0 + 1) * nb + b, rows1, gsem1, b)
            for b in range(nb):
                gwait(rows0, gsem0, b)
                scat(g0 * nb + b, rows0, ssem0, b)
            for b in range(nb):
                swait(rows0, ssem0, b)
            @pl.when(t < n_groups // 2 - 1)
            def _():
                for b in range(nb):
                    gath((g0 + 2) * nb + b, rows0, gsem0, b)
            for b in range(nb):
                gwait(rows1, gsem1, b)
                scat((g0 + 1) * nb + b, rows1, ssem1, b)
            return carry

        lax.fori_loop(0, n_groups // 2, pair, 0)
        for b in range(nb):
            swait(rows1, ssem1, b)
        plsc.subcore_barrier()
        pltpu.sync_copy(acc_sh.at[pl.ds(s * rpt, rpt)],
                        out_hbm.at[c, pl.ds(s * rpt, rpt)])

    f_k = pl.kernel(
        body,
        out_type=jax.ShapeDtypeStruct((nc, n_pad, f), jnp.float32),
        mesh=mesh,
        scratch_types=[
            pltpu.VMEM((n_chunks, _CHUNK), jnp.int32),
            pltpu.VMEM((n_chunks, _CHUNK), jnp.int32),
            pltpu.VMEM((nb, _CHUNK, f), jnp.float32),
            pltpu.VMEM((nb, _CHUNK, f), jnp.float32),
            pltpu.VMEM((128, f), jnp.float32),
            pltpu.VMEM_SHARED((n_pad, f), jnp.float32),
            pltpu.VMEM_SHARED((n_pad, f), jnp.float32),
            pltpu.SemaphoreType.DMA,
            pltpu.SemaphoreType.DMA,
            pltpu.SemaphoreType.DMA,
            pltpu.SemaphoreType.DMA,
        ],
        compiler_params=pltpu.CompilerParams(use_tc_tiling_on_sc=False),
    )
    return f_k(y, src3, dst3)


def _tc_pre(x, w1, b1, w2, b2, w3, b3, rblk):
    n_pad, din = x.shape

    def body(x_ref, w1_ref, b1_ref, w2_ref, b2_ref, w3_ref, b3_ref, s_ref):
        xv = x_ref[...]
        h1 = jnp.dot(xv, w1_ref[...], preferred_element_type=jnp.float32) + b1_ref[...]
        h2 = jnp.dot(xv, w2_ref[...], preferred_element_type=jnp.float32) + b2_ref[...]
        h3 = jnp.dot(xv, w3_ref[...], preferred_element_type=jnp.float32) + b3_ref[...]
        s_ref[...] = _relu(h1) + _relu(h2 * h3)

    wspec = pl.BlockSpec((din, _F), lambda i: (0, 0))
    bspec = pl.BlockSpec((1, _F), lambda i: (0, 0))
    return pl.pallas_call(
        body,
        grid=(n_pad // rblk,),
        in_specs=[pl.BlockSpec((rblk, din), lambda i: (i, 0)),
                  wspec, bspec, wspec, bspec, wspec, bspec],
        out_specs=pl.BlockSpec((rblk, _F), lambda i: (i, 0)),
        out_shape=jax.ShapeDtypeStruct((n_pad, _F), jnp.float32),
    )(x, w1, b1, w2, b2, w3, b3)


def _tc_mid(s_prev, ps, wc_prev, cb, w1, b1, w2, b2, w3, b3, rblk):
    n_pad = s_prev.shape[0]
    nps = len(ps)

    def body(*refs):
        s_ref = refs[0]
        p_refs = refs[1:1 + nps]
        (wcp_ref, cb_ref, w1_ref, b1_ref, w2_ref, b2_ref, w3_ref, b3_ref,
         x_ref, s2_ref) = refs[1 + nps:]
        agg = jnp.concatenate([p[0] + p[1] for p in p_refs], axis=1)
        agg = jnp.dot(agg, wcp_ref[...], preferred_element_type=jnp.float32)
        xk = s_ref[...] + _relu(agg + cb_ref[...])
        h1 = jnp.dot(xk, w1_ref[...], preferred_element_type=jnp.float32) + b1_ref[...]
        h2 = jnp.dot(xk, w2_ref[...], preferred_element_type=jnp.float32) + b2_ref[...]
        h3 = jnp.dot(xk, w3_ref[...], preferred_element_type=jnp.float32) + b3_ref[...]
        x_ref[...] = xk
        s2_ref[...] = _relu(h1) + _relu(h2 * h3)

    fin = sum(p.shape[-1] for p in ps)
    wspec = pl.BlockSpec((_F, _F), lambda i: (0, 0))
    bspec = pl.BlockSpec((1, _F), lambda i: (0, 0))

    def pspec(f):
        return pl.BlockSpec((2, rblk, f), lambda i: (0, i, 0))

    return pl.pallas_call(
        body,
        grid=(n_pad // rblk,),
        in_specs=[pl.BlockSpec((rblk, _F), lambda i: (i, 0))]
                 + [pspec(p.shape[-1]) for p in ps]
                 + [pl.BlockSpec((fin, _F), lambda i: (0, 0)),
                    bspec, wspec, bspec, wspec, bspec, wspec, bspec],
        out_specs=[pl.BlockSpec((rblk, _F), lambda i: (i, 0)),
                   pl.BlockSpec((rblk, _F), lambda i: (i, 0))],
        out_shape=[jax.ShapeDtypeStruct((n_pad, _F), jnp.float32),
                   jax.ShapeDtypeStruct((n_pad, _F), jnp.float32)],
    )(s_prev, *ps, wc_prev, cb, w1, b1, w2, b2, w3, b3)


def _tc_fin(s_prev, p, wc_prev, cb, w2, b2, rblk):
    n_pad = s_prev.shape[0]

    def body(s_ref, p_ref, wcp_ref, cb_ref, w_ref, b_ref, o_ref):
        agg = jnp.dot(p_ref[0] + p_ref[1], wcp_ref[...],
                      preferred_element_type=jnp.float32)
        xk = s_ref[...] + _relu(agg + cb_ref[...])
        o_ref[...] = jnp.dot(xk, w_ref[...],
                             preferred_element_type=jnp.float32) + b_ref[...]

    return pl.pallas_call(
        body,
        grid=(n_pad // rblk,),
        in_specs=[pl.BlockSpec((rblk, _F), lambda i: (i, 0)),
                  pl.BlockSpec((2, rblk, _F), lambda i: (0, i, 0)),
                  pl.BlockSpec((_F, _F), lambda i: (0, 0)),
                  pl.BlockSpec((1, _F), lambda i: (0, 0)),
                  pl.BlockSpec((_F, 1), lambda i: (0, 0)),
                  pl.BlockSpec((1, 1), lambda i: (0, 0))],
        out_specs=pl.BlockSpec((rblk, 1), lambda i: (i, 0)),
        out_shape=jax.ShapeDtypeStruct((n_pad, 1), jnp.float32),
    )(s_prev, p, wc_prev, cb, w2, b2)


def kernel(x, edge_index,
           conv11_w, conv11_b, conv21_w, conv21_b, conv31_w, conv31_b,
           fc11_w, fc11_b, fc12_w, fc12_b, fc13_w, fc13_b,
           fc21_w, fc21_b, fc22_w, fc22_b, fc23_w, fc23_b,
           fc31_w, fc31_b, fc32_w, fc32_b, fc33_w, fc33_b,
           fc2_w, fc2_b):
    n, din = x.shape
    e = edge_index.shape[1]
    info = plsc.get_sparse_core_info()
    nc, ns = info.num_cores, info.num_subcores
    nw = nc * ns

    n_chunks = -(-e // (nw * _CHUNK * 2 * _NB)) * (2 * _NB)
    e_pad = nw * _CHUNK * n_chunks
    n_pad = -(-n // (8 * ns)) * (8 * ns)
    if e_pad > e and n_pad == n:
        n_pad += 8 * ns
    rblk = n_pad // 4

    src = edge_index[0]
    dst = edge_index[1]
    pad = e_pad - e
    if pad:
        src = jnp.concatenate([src, jnp.zeros((pad,), jnp.int32)])
        dst = jnp.concatenate([dst, jnp.full((pad,), n, jnp.int32)])
    src3 = src.reshape(nw, n_chunks, _CHUNK)
    dst3 = dst.reshape(nw, n_chunks, _CHUNK)

    xp = jnp.pad(x, ((0, n_pad - n), (0, 0)))

    def row(b):
        return b.reshape(1, -1)

    s1 = _tc_pre(xp, fc11_w, row(fc11_b), fc12_w, row(fc12_b),
                 fc13_w, row(fc13_b), rblk)
    p1s = [_seg_sum_sc(xp[:, q * _F:(q + 1) * _F], src3, dst3, n_pad,
                       n_chunks, nc, ns, _F, _NB) for q in range(din // _F)]
    x1, s2 = _tc_mid(s1, p1s, conv11_w, row(conv11_b),
                     fc21_w, row(fc21_b), fc22_w, row(fc22_b),
                     fc23_w, row(fc23_b), rblk)
    p2 = _seg_sum_sc(x1, src3, dst3, n_pad, n_chunks, nc, ns, _F, _NB)
    x2, s3 = _tc_mid(s2, [p2], conv21_w, row(conv21_b),
                     fc31_w, row(fc31_b), fc32_w, row(fc32_b),
                     fc33_w, row(fc33_b), rblk)
    p3 = _seg_sum_sc(x2, src3, dst3, n_pad, n_chunks, nc, ns, _F, _NB)
    out = _tc_fin(s3, p3, conv31_w, row(conv31_b), fc2_w,
                  fc2_b.reshape(1, 1), rblk)
    return out[:n]
